# Initial kernel scaffold; baseline (speedup 1.0000x reference)
import functools

import jax
import jax.numpy as jnp
from jax import lax
from jax.experimental import pallas as pl
from jax.experimental.pallas import tpu as pltpu
from jax.experimental.pallas import tpu_sc as plsc

N = 10000
E = 320000
D = 128
HALF = D // 2
NC = 2
NS = 16
EB = 80
E_PER_TILE_DEG = E // (NC * NS)
E_PER_TILE = E // NS
ZCH = 80
NZCH = N // ZCH
CCH = 16
NCCH = N // CCH


def _mesh():
    return plsc.VectorSubcoreMesh(core_axis_name="c", subcore_axis_name="s")


def _deg_kernel(row_hbm, deg0_hbm, deg1_hbm, idx_v, ones_v, zb_v, deg_sh):
    c = lax.axis_index("c")
    s = lax.axis_index("s")
    wid = s * NC + c

    def fill_ones(i, _):
        ones_v[pl.ds(i * 16, 16)] = jnp.full((16,), 1.0, jnp.float32)
        return 0
    lax.fori_loop(0, EB // 16, fill_ones, 0)

    @pl.when(s == 0)
    def _():
        def fill_zb(i, _):
            zb_v[pl.ds(i * 16, 16)] = jnp.zeros((16,), jnp.float32)
            return 0
        lax.fori_loop(0, 2000 // 16, fill_zb, 0)
        def zcopy(i, _):
            pltpu.sync_copy(zb_v, deg_sh.at[pl.ds(i * 2000, 2000)])
            return 0
        lax.fori_loop(0, N // 2000, zcopy, 0)

    plsc.subcore_barrier()

    base = wid * E_PER_TILE_DEG
    def acc_body(i, _):
        off = base + i * EB
        pltpu.sync_copy(row_hbm.at[pl.ds(off, EB)], idx_v)
        pltpu.sync_copy(ones_v, deg_sh.at[idx_v], add=True)
        return 0
    lax.fori_loop(0, E_PER_TILE_DEG // EB, acc_body, 0)

    plsc.subcore_barrier()

    @pl.when(s == 0)
    def _():
        @pl.when(c == 0)
        def _():
            pltpu.sync_copy(deg_sh, deg0_hbm)
        @pl.when(c == 1)
        def _():
            pltpu.sync_copy(deg_sh, deg1_hbm)


def _run_deg(row):
    kfn = pl.kernel(
        _deg_kernel,
        out_type=[
            jax.ShapeDtypeStruct((N,), jnp.float32),
            jax.ShapeDtypeStruct((N,), jnp.float32),
        ],
        mesh=_mesh(),
        scratch_types=[
            pltpu.VMEM((EB,), jnp.int32),
            pltpu.VMEM((EB,), jnp.float32),
            pltpu.VMEM((2000,), jnp.float32),
            pltpu.VMEM_SHARED((N,), jnp.float32),
        ],
    )
    return kfn(row)


def _spmm_accumulate(z_hbm, row_hbm, col_hbm, ridx_v, cidx_v, rows_v, acc_sh, s):
    base = s * E_PER_TILE
    def body(i, _):
        off = base + i * EB
        pltpu.sync_copy(row_hbm.at[pl.ds(off, EB)], ridx_v)
        pltpu.sync_copy(col_hbm.at[pl.ds(off, EB)], cidx_v)
        pltpu.sync_copy(z_hbm.at[cidx_v], rows_v)
        pltpu.sync_copy(rows_v, acc_sh.at[ridx_v], add=True)
        return 0
    lax.fori_loop(0, E_PER_TILE // EB, body, 0)


def _spmm_combine(z_hbm, out_hbm, scale_hbm, nbuf, zbuf, abuf, obuf, acc_sh, s):
    nk = (NCCH - s + NS - 1) // NS
    def body(k, _):
        ch = s + k * NS
        r0 = ch * CCH
        pltpu.sync_copy(scale_hbm.at[pl.ds(r0, CCH)], nbuf)
        pltpu.sync_copy(z_hbm.at[pl.ds(r0, CCH)], zbuf)
        pltpu.sync_copy(acc_sh.at[pl.ds(r0, CCH)], abuf)
        for r in range(CCH):
            sc = nbuf[r]
            for j in range(HALF // 16):
                cs = pl.ds(j * 16, 16)
                obuf[r, cs] = sc * (abuf[r, cs] + zbuf[r, cs])
        pltpu.sync_copy(obuf, out_hbm.at[pl.ds(r0, CCH)])
        return 0
    lax.fori_loop(0, nk, body, 0)


def _layer_kernel(z0_hbm, z1_hbm, row_hbm, col_hbm, scale_hbm,
                  out0_hbm, out1_hbm,
                  ridx_v, cidx_v, rows_v, nbuf, zbuf, abuf, obuf, acc_sh):
    c = lax.axis_index("c")
    s = lax.axis_index("s")

    def fill_zero(r, _):
        for j in range(HALF // 16):
            rows_v[r, pl.ds(j * 16, 16)] = jnp.zeros((16,), jnp.float32)
        return 0
    lax.fori_loop(0, ZCH, fill_zero, 0)
    nz = (NZCH - s + NS - 1) // NS
    def zero_body(k, _):
        ch = s + k * NS
        pltpu.sync_copy(rows_v, acc_sh.at[pl.ds(ch * ZCH, ZCH)])
        return 0
    lax.fori_loop(0, nz, zero_body, 0)

    plsc.subcore_barrier()

    @pl.when(c == 0)
    def _():
        _spmm_accumulate(z0_hbm, row_hbm, col_hbm, ridx_v, cidx_v, rows_v, acc_sh, s)
    @pl.when(c == 1)
    def _():
        _spmm_accumulate(z1_hbm, row_hbm, col_hbm, ridx_v, cidx_v, rows_v, acc_sh, s)

    plsc.subcore_barrier()

    @pl.when(c == 0)
    def _():
        _spmm_combine(z0_hbm, out0_hbm, scale_hbm, nbuf, zbuf, abuf, obuf, acc_sh, s)
    @pl.when(c == 1)
    def _():
        _spmm_combine(z1_hbm, out1_hbm, scale_hbm, nbuf, zbuf, abuf, obuf, acc_sh, s)


def _run_layer(z0, z1, row, col, scale):
    kfn = pl.kernel(
        _layer_kernel,
        out_type=[
            jax.ShapeDtypeStruct((N, HALF), jnp.float32),
            jax.ShapeDtypeStruct((N, HALF), jnp.float32),
        ],
        mesh=_mesh(),
        scratch_types=[
            pltpu.VMEM((EB,), jnp.int32),
            pltpu.VMEM((EB,), jnp.int32),
            pltpu.VMEM((EB, HALF), jnp.float32),
            pltpu.VMEM((CCH,), jnp.float32),
            pltpu.VMEM((CCH, HALF), jnp.float32),
            pltpu.VMEM((CCH, HALF), jnp.float32),
            pltpu.VMEM((CCH, HALF), jnp.float32),
            pltpu.VMEM_SHARED((N, HALF), jnp.float32),
        ],
    )
    return kfn(z0, z1, row, col, scale)


def _tc_prep_kernel(x_ref, w_ref, b_ref, d0_ref, d1_ref,
                    z0_ref, z1_ref, n_ref, n2_ref):
    h = jnp.dot(x_ref[...], w_ref[...], preferred_element_type=jnp.float32)
    h = h + b_ref[...]
    norm = lax.rsqrt(1.0 + d0_ref[...] + d1_ref[...])
    z = norm * h
    z0_ref[...] = z[:, :HALF]
    z1_ref[...] = z[:, HALF:]
    n_ref[...] = norm
    n2_ref[...] = norm * norm


def _run_prep(x, W, b, deg0, deg1):
    return pl.pallas_call(
        _tc_prep_kernel,
        out_shape=[
            jax.ShapeDtypeStruct((N, HALF), jnp.float32),
            jax.ShapeDtypeStruct((N, HALF), jnp.float32),
            jax.ShapeDtypeStruct((N, 1), jnp.float32),
            jax.ShapeDtypeStruct((N, 1), jnp.float32),
        ],
    )(x, W, b.reshape(1, D), deg0.reshape(N, 1), deg1.reshape(N, 1))


def kernel(x, edge_index, W, b):
    row = edge_index[0]
    col = edge_index[1]
    deg0, deg1 = _run_deg(row)
    z0, z1, norm, norm2 = _run_prep(x, W, b, deg0, deg1)
    sc_n = norm.reshape(N)
    sc_n2 = norm2.reshape(N)
    z0, z1 = _run_layer(z0, z1, row, col, sc_n2)
    z0, z1 = _run_layer(z0, z1, row, col, sc_n2)
    o0, o1 = _run_layer(z0, z1, row, col, sc_n)
    return jnp.concatenate([o0, o1], axis=1)


# speedup vs baseline: 4.5074x; 4.5074x over previous
import functools

import jax
import jax.numpy as jnp
from jax import lax
from jax.experimental import pallas as pl
from jax.experimental.pallas import tpu as pltpu
from jax.experimental.pallas import tpu_sc as plsc

N = 10000
E = 320000
D = 128
NC = 2
NS = 16
EB = 80
E_PER_TILE = E // (NC * NS)
ZCH = 80
NZCH = N // ZCH


def _mesh():
    return plsc.VectorSubcoreMesh(core_axis_name="c", subcore_axis_name="s")


def _deg_kernel(row_hbm, deg0_hbm, deg1_hbm, idx_v, ones_v, zb_v, deg_sh):
    c = lax.axis_index("c")
    s = lax.axis_index("s")
    wid = s * NC + c

    def fill_ones(i, _):
        ones_v[pl.ds(i * 16, 16)] = jnp.full((16,), 1.0, jnp.float32)
        return 0
    lax.fori_loop(0, EB // 16, fill_ones, 0)

    @pl.when(s == 0)
    def _():
        def fill_zb(i, _):
            zb_v[pl.ds(i * 16, 16)] = jnp.zeros((16,), jnp.float32)
            return 0
        lax.fori_loop(0, 2000 // 16, fill_zb, 0)
        def zcopy(i, _):
            pltpu.sync_copy(zb_v, deg_sh.at[pl.ds(i * 2000, 2000)])
            return 0
        lax.fori_loop(0, N // 2000, zcopy, 0)

    plsc.subcore_barrier()

    base = wid * E_PER_TILE
    def acc_body(i, _):
        off = base + i * EB
        pltpu.sync_copy(row_hbm.at[pl.ds(off, EB)], idx_v)
        pltpu.sync_copy(ones_v, deg_sh.at[idx_v], add=True)
        return 0
    lax.fori_loop(0, E_PER_TILE // EB, acc_body, 0)

    plsc.subcore_barrier()

    @pl.when(s == 0)
    def _():
        @pl.when(c == 0)
        def _():
            pltpu.sync_copy(deg_sh, deg0_hbm)
        @pl.when(c == 1)
        def _():
            pltpu.sync_copy(deg_sh, deg1_hbm)


def _run_deg(row):
    kfn = pl.kernel(
        _deg_kernel,
        out_type=[
            jax.ShapeDtypeStruct((N,), jnp.float32),
            jax.ShapeDtypeStruct((N,), jnp.float32),
        ],
        mesh=_mesh(),
        scratch_types=[
            pltpu.VMEM((EB,), jnp.int32),
            pltpu.VMEM((EB,), jnp.float32),
            pltpu.VMEM((2000,), jnp.float32),
            pltpu.VMEM_SHARED((N,), jnp.float32),
        ],
    )
    return kfn(row)


def _spmm_kernel(z_hbm, row_hbm, col_hbm, p0_hbm, p1_hbm,
                 ridx_v, cidx_v, rows_v, acc_sh):
    c = lax.axis_index("c")
    s = lax.axis_index("s")
    wid = s * NC + c

    def fill_zero(r, _):
        for j in range(D // 16):
            rows_v[r, pl.ds(j * 16, 16)] = jnp.zeros((16,), jnp.float32)
        return 0
    lax.fori_loop(0, ZCH, fill_zero, 0)
    nz = (NZCH - s + NS - 1) // NS
    def zero_body(k, _):
        ch = s + k * NS
        pltpu.sync_copy(rows_v, acc_sh.at[pl.ds(ch * ZCH, ZCH)])
        return 0
    lax.fori_loop(0, nz, zero_body, 0)

    plsc.subcore_barrier()

    base = wid * E_PER_TILE
    def body(i, _):
        off = base + i * EB
        pltpu.sync_copy(row_hbm.at[pl.ds(off, EB)], ridx_v)
        pltpu.sync_copy(col_hbm.at[pl.ds(off, EB)], cidx_v)
        pltpu.sync_copy(z_hbm.at[cidx_v], rows_v)
        pltpu.sync_copy(rows_v, acc_sh.at[ridx_v], add=True)
        return 0
    lax.fori_loop(0, E_PER_TILE // EB, body, 0)

    plsc.subcore_barrier()

    def dump(k, _):
        ch = s + k * NS
        r0 = ch * ZCH
        @pl.when(c == 0)
        def _():
            pltpu.sync_copy(acc_sh.at[pl.ds(r0, ZCH)], p0_hbm.at[pl.ds(r0, ZCH)])
        @pl.when(c == 1)
        def _():
            pltpu.sync_copy(acc_sh.at[pl.ds(r0, ZCH)], p1_hbm.at[pl.ds(r0, ZCH)])
        return 0
    lax.fori_loop(0, nz, dump, 0)


def _run_spmm(z, row, col):
    kfn = pl.kernel(
        _spmm_kernel,
        out_type=[
            jax.ShapeDtypeStruct((N, D), jnp.float32),
            jax.ShapeDtypeStruct((N, D), jnp.float32),
        ],
        mesh=_mesh(),
        scratch_types=[
            pltpu.VMEM((EB,), jnp.int32),
            pltpu.VMEM((EB,), jnp.int32),
            pltpu.VMEM((EB, D), jnp.float32),
            pltpu.VMEM_SHARED((N, D), jnp.float32),
        ],
    )
    return kfn(z, row, col)


def _tc_prep_kernel(x_ref, w_ref, b_ref, d0_ref, d1_ref,
                    z_ref, n_ref, n2_ref):
    h = jnp.dot(x_ref[...], w_ref[...], preferred_element_type=jnp.float32)
    h = h + b_ref[...]
    norm = lax.rsqrt(1.0 + d0_ref[...] + d1_ref[...])
    z_ref[...] = norm * h
    n_ref[...] = norm
    n2_ref[...] = norm * norm


def _run_prep(x, W, b, deg0, deg1):
    return pl.pallas_call(
        _tc_prep_kernel,
        out_shape=[
            jax.ShapeDtypeStruct((N, D), jnp.float32),
            jax.ShapeDtypeStruct((N, 1), jnp.float32),
            jax.ShapeDtypeStruct((N, 1), jnp.float32),
        ],
    )(x, W, b.reshape(1, D), deg0.reshape(N, 1), deg1.reshape(N, 1))


def _tc_combine_kernel(p0_ref, p1_ref, z_ref, s_ref, o_ref):
    o_ref[...] = s_ref[...] * (p0_ref[...] + p1_ref[...] + z_ref[...])


def _run_combine(p0, p1, z, scale):
    return pl.pallas_call(
        _tc_combine_kernel,
        out_shape=jax.ShapeDtypeStruct((N, D), jnp.float32),
    )(p0, p1, z, scale)


def kernel(x, edge_index, W, b):
    row = edge_index[0]
    col = edge_index[1]
    deg0, deg1 = _run_deg(row)
    z, norm, norm2 = _run_prep(x, W, b, deg0, deg1)
    for scale in (norm2, norm2, norm):
        p0, p1 = _run_spmm(z, row, col)
        z = _run_combine(p0, p1, z, scale)
    return z


# speedup vs baseline: 9.9578x; 2.2092x over previous
import functools

import jax
import jax.numpy as jnp
from jax import lax
from jax.experimental import pallas as pl
from jax.experimental.pallas import tpu as pltpu
from jax.experimental.pallas import tpu_sc as plsc

N = 10000
E = 320000
D = 128
NC = 2
NS = 16
EB = 80
E_PER_TILE = E // (NC * NS)
ZCH = 80
NZCH = N // ZCH


def _mesh():
    return plsc.VectorSubcoreMesh(core_axis_name="c", subcore_axis_name="s")


def _deg_kernel(row_hbm, deg0_hbm, deg1_hbm, idx_v, ones_v, zb_v, deg_sh):
    c = lax.axis_index("c")
    s = lax.axis_index("s")
    wid = s * NC + c

    def fill_ones(i, _):
        ones_v[pl.ds(i * 16, 16)] = jnp.full((16,), 1.0, jnp.float32)
        return 0
    lax.fori_loop(0, EB // 16, fill_ones, 0)

    @pl.when(s == 0)
    def _():
        def fill_zb(i, _):
            zb_v[pl.ds(i * 16, 16)] = jnp.zeros((16,), jnp.float32)
            return 0
        lax.fori_loop(0, 2000 // 16, fill_zb, 0)
        def zcopy(i, _):
            pltpu.sync_copy(zb_v, deg_sh.at[pl.ds(i * 2000, 2000)])
            return 0
        lax.fori_loop(0, N // 2000, zcopy, 0)

    plsc.subcore_barrier()

    base = wid * E_PER_TILE
    def acc_body(i, _):
        off = base + i * EB
        pltpu.sync_copy(row_hbm.at[pl.ds(off, EB)], idx_v)
        pltpu.sync_copy(ones_v, deg_sh.at[idx_v], add=True)
        return 0
    lax.fori_loop(0, E_PER_TILE // EB, acc_body, 0)

    plsc.subcore_barrier()

    @pl.when(s == 0)
    def _():
        @pl.when(c == 0)
        def _():
            pltpu.sync_copy(deg_sh, deg0_hbm)
        @pl.when(c == 1)
        def _():
            pltpu.sync_copy(deg_sh, deg1_hbm)


def _run_deg(row):
    kfn = pl.kernel(
        _deg_kernel,
        out_type=[
            jax.ShapeDtypeStruct((N,), jnp.float32),
            jax.ShapeDtypeStruct((N,), jnp.float32),
        ],
        mesh=_mesh(),
        scratch_types=[
            pltpu.VMEM((EB,), jnp.int32),
            pltpu.VMEM((EB,), jnp.float32),
            pltpu.VMEM((2000,), jnp.float32),
            pltpu.VMEM_SHARED((N,), jnp.float32),
        ],
    )
    return kfn(row)


NSLOT = 4
BB = 80
NB = E_PER_TILE // BB
MAIN_G = NB // NSLOT - 1


def _spmm_kernel(z_hbm, row_hbm, col_hbm, p0_hbm, p1_hbm,
                 ridx_v, cidx_v, rows_v,
                 sem_i, sem_g, sem_s, acc_sh):
    c = lax.axis_index("c")
    s = lax.axis_index("s")
    base = c * (E // NC) + s * E_PER_TILE

    def fill_zero(r, _):
        for j in range(D // 16):
            rows_v[0, r, pl.ds(j * 16, 16)] = jnp.zeros((16,), jnp.float32)
        return 0
    lax.fori_loop(0, ZCH, fill_zero, 0)
    nz = (NZCH - s + NS - 1) // NS
    def zero_body(k, _):
        ch = s + k * NS
        pltpu.sync_copy(rows_v.at[0], acc_sh.at[pl.ds(ch * ZCH, ZCH)])
        return 0
    lax.fori_loop(0, nz, zero_body, 0)

    plsc.subcore_barrier()

    def idx_load(k, b, h):
        off = base + k * BB
        pltpu.async_copy(row_hbm.at[pl.ds(off, BB)], ridx_v.at[h, b], sem_i.at[b])
        pltpu.async_copy(col_hbm.at[pl.ds(off, BB)], cidx_v.at[b], sem_i.at[b])

    def idx_wait(b, h):
        pltpu.make_async_copy(row_hbm.at[pl.ds(0, BB)], ridx_v.at[h, b], sem_i.at[b]).wait()
        pltpu.make_async_copy(col_hbm.at[pl.ds(0, BB)], cidx_v.at[b], sem_i.at[b]).wait()

    def gather(b):
        pltpu.async_copy(z_hbm.at[cidx_v.at[b]], rows_v.at[b], sem_g.at[b])

    def gather_wait(b):
        pltpu.make_async_copy(z_hbm.at[cidx_v.at[b]], rows_v.at[b], sem_g.at[b]).wait()

    def scatter(b, h):
        pltpu.async_copy(rows_v.at[b], acc_sh.at[ridx_v.at[h, b]], sem_s.at[b], add=True)

    def scatter_wait(b, h):
        pltpu.make_async_copy(rows_v.at[b], acc_sh.at[ridx_v.at[h, b]], sem_s.at[b]).wait()

    for b in range(NSLOT):
        idx_load(b, b, 0)
    for b in range(NSLOT):
        idx_wait(b, 0)
        gather(b)

    def body(gg, _):
        for gpar in (0, 1):
            g = 2 * gg + gpar
            for b in range(NSLOT):
                gather_wait(b)
                idx_load(g * NSLOT + b + NSLOT, b, 1 - gpar)
                scatter(b, gpar)
            for b in range(NSLOT):
                idx_wait(b, 1 - gpar)
                scatter_wait(b, gpar)
                gather(b)
        return 0
    lax.fori_loop(0, MAIN_G // 2, body, 0)

    for b in range(NSLOT):
        gather_wait(b)
        scatter(b, 0)
    for b in range(NSLOT):
        scatter_wait(b, 0)

    idx_load(NB - 1, 0, 0)
    idx_wait(0, 0)
    gather(0)
    gather_wait(0)
    scatter(0, 0)
    scatter_wait(0, 0)

    plsc.subcore_barrier()

    def dump(k, _):
        ch = s + k * NS
        r0 = ch * ZCH
        @pl.when(c == 0)
        def _():
            pltpu.sync_copy(acc_sh.at[pl.ds(r0, ZCH)], p0_hbm.at[pl.ds(r0, ZCH)])
        @pl.when(c == 1)
        def _():
            pltpu.sync_copy(acc_sh.at[pl.ds(r0, ZCH)], p1_hbm.at[pl.ds(r0, ZCH)])
        return 0
    lax.fori_loop(0, nz, dump, 0)


def _run_spmm(z, row, col):
    kfn = pl.kernel(
        _spmm_kernel,
        out_type=[
            jax.ShapeDtypeStruct((N, D), jnp.float32),
            jax.ShapeDtypeStruct((N, D), jnp.float32),
        ],
        mesh=_mesh(),
        scratch_types=[
            pltpu.VMEM((2, NSLOT, BB), jnp.int32),
            pltpu.VMEM((NSLOT, BB), jnp.int32),
            pltpu.VMEM((NSLOT, BB, D), jnp.float32),
            pltpu.SemaphoreType.DMA((NSLOT,)),
            pltpu.SemaphoreType.DMA((NSLOT,)),
            pltpu.SemaphoreType.DMA((NSLOT,)),
            pltpu.VMEM_SHARED((N, D), jnp.float32),
        ],
    )
    return kfn(z, row, col)


def _tc_prep_kernel(x_ref, w_ref, b_ref, d0_ref, d1_ref,
                    z_ref, n_ref, n2_ref):
    h = jnp.dot(x_ref[...], w_ref[...], preferred_element_type=jnp.float32)
    h = h + b_ref[...]
    norm = lax.rsqrt(1.0 + d0_ref[...] + d1_ref[...])
    z_ref[...] = norm * h
    n_ref[...] = norm
    n2_ref[...] = norm * norm


def _run_prep(x, W, b, deg0, deg1):
    return pl.pallas_call(
        _tc_prep_kernel,
        out_shape=[
            jax.ShapeDtypeStruct((N, D), jnp.float32),
            jax.ShapeDtypeStruct((N, 1), jnp.float32),
            jax.ShapeDtypeStruct((N, 1), jnp.float32),
        ],
    )(x, W, b.reshape(1, D), deg0.reshape(N, 1), deg1.reshape(N, 1))


def _tc_combine_kernel(p0_ref, p1_ref, z_ref, s_ref, o_ref):
    o_ref[...] = s_ref[...] * (p0_ref[...] + p1_ref[...] + z_ref[...])


def _run_combine(p0, p1, z, scale):
    return pl.pallas_call(
        _tc_combine_kernel,
        out_shape=jax.ShapeDtypeStruct((N, D), jnp.float32),
    )(p0, p1, z, scale)


def kernel(x, edge_index, W, b):
    row = edge_index[0]
    col = edge_index[1]
    deg0, deg1 = _run_deg(row)
    z, norm, norm2 = _run_prep(x, W, b, deg0, deg1)
    for scale in (norm2, norm2, norm):
        p0, p1 = _run_spmm(z, row, col)
        z = _run_combine(p0, p1, z, scale)
    return z


# speedup vs baseline: 11.2071x; 1.1255x over previous
import functools

import jax
import jax.numpy as jnp
from jax import lax
from jax.experimental import pallas as pl
from jax.experimental.pallas import tpu as pltpu
from jax.experimental.pallas import tpu_sc as plsc

N = 10000
E = 320000
D = 128
NC = 2
NS = 16
EB = 80
E_PER_TILE = E // (NC * NS)
ZCH = 80
NZCH = N // ZCH


def _mesh():
    return plsc.VectorSubcoreMesh(core_axis_name="c", subcore_axis_name="s")


DSLOT = 8
DNB = E // (NC * NS) // EB
DMAIN_G = 14


def _deg_kernel(row_hbm, deg0_hbm, deg1_hbm, idx_v, ones_v, zb_v,
                sem_i, sem_s, deg_sh):
    c = lax.axis_index("c")
    s = lax.axis_index("s")
    wid = s * NC + c

    def fill_ones(i, _):
        ones_v[pl.ds(i * 16, 16)] = jnp.full((16,), 1.0, jnp.float32)
        return 0
    lax.fori_loop(0, EB // 16, fill_ones, 0)

    @pl.when(s == 0)
    def _():
        def fill_zb(i, _):
            zb_v[pl.ds(i * 16, 16)] = jnp.zeros((16,), jnp.float32)
            return 0
        lax.fori_loop(0, 2000 // 16, fill_zb, 0)
        def zcopy(i, _):
            pltpu.sync_copy(zb_v, deg_sh.at[pl.ds(i * 2000, 2000)])
            return 0
        lax.fori_loop(0, N // 2000, zcopy, 0)

    plsc.subcore_barrier()

    base = wid * E_PER_TILE

    def idx_load(k, b):
        pltpu.async_copy(row_hbm.at[pl.ds(base + k * EB, EB)], idx_v.at[b],
                         sem_i.at[b])

    def idx_wait(b):
        pltpu.make_async_copy(row_hbm.at[pl.ds(0, EB)], idx_v.at[b],
                              sem_i.at[b]).wait()

    def scat(b):
        pltpu.async_copy(ones_v, deg_sh.at[idx_v.at[b]], sem_s.at[b], add=True)

    def scat_wait(b):
        pltpu.make_async_copy(ones_v, deg_sh.at[idx_v.at[b]], sem_s.at[b]).wait()

    for b in range(DSLOT):
        idx_load(b, b)

    def acc_body(g, _):
        for b in range(DSLOT):
            idx_wait(b)
            scat(b)
        for b in range(DSLOT):
            scat_wait(b)
            idx_load((g + 1) * DSLOT + b, b)
        return 0
    lax.fori_loop(0, DMAIN_G, acc_body, 0)

    for b in range(DSLOT):
        idx_wait(b)
        scat(b)
    for b in range(DSLOT):
        scat_wait(b)
    for j, k in enumerate(range((DMAIN_G + 1) * DSLOT, DNB)):
        idx_load(k, j)
    for j, k in enumerate(range((DMAIN_G + 1) * DSLOT, DNB)):
        idx_wait(j)
        scat(j)
    for j, k in enumerate(range((DMAIN_G + 1) * DSLOT, DNB)):
        scat_wait(j)

    plsc.subcore_barrier()

    @pl.when(s == 0)
    def _():
        @pl.when(c == 0)
        def _():
            pltpu.sync_copy(deg_sh, deg0_hbm)
        @pl.when(c == 1)
        def _():
            pltpu.sync_copy(deg_sh, deg1_hbm)


def _run_deg(row):
    kfn = pl.kernel(
        _deg_kernel,
        out_type=[
            jax.ShapeDtypeStruct((N,), jnp.float32),
            jax.ShapeDtypeStruct((N,), jnp.float32),
        ],
        mesh=_mesh(),
        scratch_types=[
            pltpu.VMEM((DSLOT, EB), jnp.int32),
            pltpu.VMEM((EB,), jnp.float32),
            pltpu.VMEM((2000,), jnp.float32),
            pltpu.SemaphoreType.DMA((DSLOT,)),
            pltpu.SemaphoreType.DMA((DSLOT,)),
            pltpu.VMEM_SHARED((N,), jnp.float32),
        ],
    )
    return kfn(row)


NSLOT = 4
BB = 80
NB = E_PER_TILE // BB
MAIN_G = NB // NSLOT - 1


def _spmm_kernel(z_hbm, row_hbm, col_hbm, p0_hbm, p1_hbm,
                 ridx_v, cidx_v, rows_v,
                 sem_i, sem_g, sem_s, acc_sh):
    c = lax.axis_index("c")
    s = lax.axis_index("s")
    base = c * (E // NC) + s * E_PER_TILE

    def fill_zero(r, _):
        for j in range(D // 16):
            rows_v[0, r, pl.ds(j * 16, 16)] = jnp.zeros((16,), jnp.float32)
        return 0
    lax.fori_loop(0, ZCH, fill_zero, 0)
    nz = (NZCH - s + NS - 1) // NS
    def zero_body(k, _):
        ch = s + k * NS
        pltpu.sync_copy(rows_v.at[0], acc_sh.at[pl.ds(ch * ZCH, ZCH)])
        return 0
    lax.fori_loop(0, nz, zero_body, 0)

    plsc.subcore_barrier()

    def idx_load(k, b, h):
        off = base + k * BB
        pltpu.async_copy(row_hbm.at[pl.ds(off, BB)], ridx_v.at[h, b], sem_i.at[b])
        pltpu.async_copy(col_hbm.at[pl.ds(off, BB)], cidx_v.at[b], sem_i.at[b])

    def idx_wait(b, h):
        pltpu.make_async_copy(row_hbm.at[pl.ds(0, BB)], ridx_v.at[h, b], sem_i.at[b]).wait()
        pltpu.make_async_copy(col_hbm.at[pl.ds(0, BB)], cidx_v.at[b], sem_i.at[b]).wait()

    def gather(b):
        pltpu.async_copy(z_hbm.at[cidx_v.at[b]], rows_v.at[b], sem_g.at[b])

    def gather_wait(b):
        pltpu.make_async_copy(z_hbm.at[cidx_v.at[b]], rows_v.at[b], sem_g.at[b]).wait()

    def scatter(b, h):
        pltpu.async_copy(rows_v.at[b], acc_sh.at[ridx_v.at[h, b]], sem_s.at[b], add=True)

    def scatter_wait(b, h):
        pltpu.make_async_copy(rows_v.at[b], acc_sh.at[ridx_v.at[h, b]], sem_s.at[b]).wait()

    for b in range(NSLOT):
        idx_load(b, b, 0)
    for b in range(NSLOT):
        idx_wait(b, 0)
        gather(b)

    def body(gg, _):
        for gpar in (0, 1):
            g = 2 * gg + gpar
            for b in range(NSLOT):
                gather_wait(b)
                idx_load(g * NSLOT + b + NSLOT, b, 1 - gpar)
                scatter(b, gpar)
            for b in range(NSLOT):
                idx_wait(b, 1 - gpar)
                scatter_wait(b, gpar)
                gather(b)
        return 0
    lax.fori_loop(0, MAIN_G // 2, body, 0)

    for b in range(NSLOT):
        gather_wait(b)
        scatter(b, 0)
    for b in range(NSLOT):
        scatter_wait(b, 0)

    idx_load(NB - 1, 0, 0)
    idx_wait(0, 0)
    gather(0)
    gather_wait(0)
    scatter(0, 0)
    scatter_wait(0, 0)

    plsc.subcore_barrier()

    def dump(k, _):
        ch = s + k * NS
        r0 = ch * ZCH
        @pl.when(c == 0)
        def _():
            pltpu.sync_copy(acc_sh.at[pl.ds(r0, ZCH)], p0_hbm.at[pl.ds(r0, ZCH)])
        @pl.when(c == 1)
        def _():
            pltpu.sync_copy(acc_sh.at[pl.ds(r0, ZCH)], p1_hbm.at[pl.ds(r0, ZCH)])
        return 0
    lax.fori_loop(0, nz, dump, 0)


def _run_spmm(z, row, col):
    kfn = pl.kernel(
        _spmm_kernel,
        out_type=[
            jax.ShapeDtypeStruct((N, D), jnp.float32),
            jax.ShapeDtypeStruct((N, D), jnp.float32),
        ],
        mesh=_mesh(),
        scratch_types=[
            pltpu.VMEM((2, NSLOT, BB), jnp.int32),
            pltpu.VMEM((NSLOT, BB), jnp.int32),
            pltpu.VMEM((NSLOT, BB, D), jnp.float32),
            pltpu.SemaphoreType.DMA((NSLOT,)),
            pltpu.SemaphoreType.DMA((NSLOT,)),
            pltpu.SemaphoreType.DMA((NSLOT,)),
            pltpu.VMEM_SHARED((N, D), jnp.float32),
        ],
    )
    return kfn(z, row, col)


def _tc_prep_kernel(x_ref, w_ref, b_ref, d0_ref, d1_ref,
                    z_ref, n_ref, n2_ref):
    h = jnp.dot(x_ref[...], w_ref[...], preferred_element_type=jnp.float32)
    h = h + b_ref[...]
    norm = lax.rsqrt(1.0 + d0_ref[...] + d1_ref[...])
    z_ref[...] = norm * h
    n_ref[...] = norm
    n2_ref[...] = norm * norm


def _run_prep(x, W, b, deg0, deg1):
    return pl.pallas_call(
        _tc_prep_kernel,
        out_shape=[
            jax.ShapeDtypeStruct((N, D), jnp.float32),
            jax.ShapeDtypeStruct((N, 1), jnp.float32),
            jax.ShapeDtypeStruct((N, 1), jnp.float32),
        ],
    )(x, W, b.reshape(1, D), deg0.reshape(N, 1), deg1.reshape(N, 1))


def _tc_combine_kernel(p0_ref, p1_ref, z_ref, s_ref, o_ref):
    o_ref[...] = s_ref[...] * (p0_ref[...] + p1_ref[...] + z_ref[...])


def _run_combine(p0, p1, z, scale):
    return pl.pallas_call(
        _tc_combine_kernel,
        out_shape=jax.ShapeDtypeStruct((N, D), jnp.float32),
    )(p0, p1, z, scale)


def kernel(x, edge_index, W, b):
    row = edge_index[0]
    col = edge_index[1]
    deg0, deg1 = _run_deg(row)
    z, norm, norm2 = _run_prep(x, W, b, deg0, deg1)
    for scale in (norm2, norm2, norm):
        p0, p1 = _run_spmm(z, row, col)
        z = _run_combine(p0, p1, z, scale)
    return z
